# baseline (speedup 1.0000x reference)
import jax
import jax.numpy as jnp
from jax.experimental import pallas as pl
from jax.experimental.pallas import tpu as pltpu

N = 4096
D = 128
K = 10
THETA = 5.0
BLK = 256
NEG = -3.0e38
LEAF_W = 512


def _normalize_kernel(h_ref, hn_ref):
    h = h_ref[...]
    norm = jnp.sqrt(jnp.sum(h * h, axis=1, keepdims=True))
    hn_ref[...] = h / jnp.maximum(norm, 1e-8)


def _distinct_maxima(s, k):
    out = []
    m = jnp.max(s, axis=1, keepdims=True)
    out.append(m)
    for _ in range(k - 1):
        m = jnp.max(jnp.where(s < m, s, NEG), axis=1, keepdims=True)
        out.append(m)
    return out


def _topk_candidates(s, k):
    if s.shape[1] > LEAF_W and k >= 2:
        half = s.shape[1] // 2
        a = s[:, :half]
        b = s[:, half:]
        hi = jnp.maximum(a, b)
        lo = jnp.minimum(a, b)
        return _topk_candidates(hi, k) + _topk_candidates(lo, (k + 1) // 2)
    return _distinct_maxima(s, k)


def _fused_kernel(hn_ref, hnb_ref, h_ref, o_ref):
    hn = hn_ref[...]
    h = h_ref[...]
    hn_blk = hnb_ref[...]

    s = jnp.dot(hn_blk, hn.T, preferred_element_type=jnp.float32)

    cand = jnp.concatenate(_topk_candidates(s, K), axis=1)
    t = _distinct_maxima(cand, K)[K - 1]

    e = jnp.where(s >= t, jnp.exp(THETA * s), 0.0)
    denom = jnp.sum(e, axis=1, keepdims=True)
    o_ref[...] = jnp.dot(e, h, preferred_element_type=jnp.float32) / denom


def kernel(h):
    hn = pl.pallas_call(
        _normalize_kernel,
        in_specs=[pl.BlockSpec((N, D), lambda: (0, 0))],
        out_specs=pl.BlockSpec((N, D), lambda: (0, 0)),
        out_shape=jax.ShapeDtypeStruct((N, D), jnp.float32),
    )(h)
    return pl.pallas_call(
        _fused_kernel,
        grid=(N // BLK,),
        in_specs=[
            pl.BlockSpec((N, D), lambda i: (0, 0)),
            pl.BlockSpec((BLK, D), lambda i: (i, 0)),
            pl.BlockSpec((N, D), lambda i: (0, 0)),
        ],
        out_specs=pl.BlockSpec((BLK, D), lambda i: (i, 0)),
        out_shape=jax.ShapeDtypeStruct((N, D), jnp.float32),
        compiler_params=pltpu.CompilerParams(
            dimension_semantics=("parallel",),
        ),
    )(hn, hn, h)


# speedup vs baseline: 1.0885x; 1.0885x over previous
import jax
import jax.numpy as jnp
from jax.experimental import pallas as pl
from jax.experimental.pallas import tpu as pltpu

N = 4096
D = 128
K = 10
THETA = 5.0
BLK = 512
NEG = -3.0e38
LEAF_W = 512


def _normalize_kernel(h_ref, hn_ref):
    h = h_ref[...]
    norm = jnp.sqrt(jnp.sum(h * h, axis=1, keepdims=True))
    hn_ref[...] = h / jnp.maximum(norm, 1e-8)


def _distinct_maxima(s, k):
    out = []
    m = jnp.max(s, axis=1, keepdims=True)
    out.append(m)
    for _ in range(k - 1):
        m = jnp.max(jnp.where(s < m, s, NEG), axis=1, keepdims=True)
        out.append(m)
    return out


def _topk_candidates(s, k):
    if s.shape[1] > LEAF_W and k >= 2:
        half = s.shape[1] // 2
        a = s[:, :half]
        b = s[:, half:]
        hi = jnp.maximum(a, b)
        lo = jnp.minimum(a, b)
        return _topk_candidates(hi, k) + _topk_candidates(lo, (k + 1) // 2)
    return _distinct_maxima(s, k)


def _fused_kernel(hn_ref, hnb_ref, h_ref, o_ref):
    hn = hn_ref[...]
    h = h_ref[...]
    hn_blk = hnb_ref[...]

    s = jnp.dot(hn_blk, hn.T, preferred_element_type=jnp.float32)

    cand = jnp.concatenate(_topk_candidates(s, K), axis=1)
    t = _distinct_maxima(cand, K)[K - 1]

    e = jnp.where(s >= t, jnp.exp(THETA * s), 0.0)
    denom = jnp.sum(e, axis=1, keepdims=True)
    o_ref[...] = jnp.dot(e, h, preferred_element_type=jnp.float32) / denom


def kernel(h):
    hn = pl.pallas_call(
        _normalize_kernel,
        in_specs=[pl.BlockSpec((N, D), lambda: (0, 0))],
        out_specs=pl.BlockSpec((N, D), lambda: (0, 0)),
        out_shape=jax.ShapeDtypeStruct((N, D), jnp.float32),
    )(h)
    return pl.pallas_call(
        _fused_kernel,
        grid=(N // BLK,),
        in_specs=[
            pl.BlockSpec((N, D), lambda i: (0, 0)),
            pl.BlockSpec((BLK, D), lambda i: (i, 0)),
            pl.BlockSpec((N, D), lambda i: (0, 0)),
        ],
        out_specs=pl.BlockSpec((BLK, D), lambda i: (i, 0)),
        out_shape=jax.ShapeDtypeStruct((N, D), jnp.float32),
        compiler_params=pltpu.CompilerParams(
            dimension_semantics=("parallel",),
        ),
    )(hn, hn, h)


# speedup vs baseline: 1.1209x; 1.0298x over previous
import jax
import jax.numpy as jnp
from jax.experimental import pallas as pl
from jax.experimental.pallas import tpu as pltpu

N = 4096
D = 128
K = 10
THETA = 5.0
BLK = 1024
NEG = -3.0e38
LEAF_W = 512


def _normalize_kernel(h_ref, hn_ref):
    h = h_ref[...]
    norm = jnp.sqrt(jnp.sum(h * h, axis=1, keepdims=True))
    hn_ref[...] = h / jnp.maximum(norm, 1e-8)


def _distinct_maxima(s, k):
    out = []
    m = jnp.max(s, axis=1, keepdims=True)
    out.append(m)
    for _ in range(k - 1):
        m = jnp.max(jnp.where(s < m, s, NEG), axis=1, keepdims=True)
        out.append(m)
    return out


def _topk_candidates(s, k):
    if s.shape[1] > LEAF_W and k >= 2:
        half = s.shape[1] // 2
        a = s[:, :half]
        b = s[:, half:]
        hi = jnp.maximum(a, b)
        lo = jnp.minimum(a, b)
        return _topk_candidates(hi, k) + _topk_candidates(lo, (k + 1) // 2)
    return _distinct_maxima(s, k)


def _fused_kernel(hn_ref, hnb_ref, h_ref, o_ref):
    hn = hn_ref[...]
    h = h_ref[...]
    hn_blk = hnb_ref[...]

    s = jnp.dot(hn_blk, hn.T, preferred_element_type=jnp.float32)

    cand = jnp.concatenate(_topk_candidates(s, K), axis=1)
    t = _distinct_maxima(cand, K)[K - 1]

    e = jnp.where(s >= t, jnp.exp(THETA * s), 0.0)
    denom = jnp.sum(e, axis=1, keepdims=True)
    o_ref[...] = jnp.dot(e, h, preferred_element_type=jnp.float32) / denom


def kernel(h):
    hn = pl.pallas_call(
        _normalize_kernel,
        in_specs=[pl.BlockSpec((N, D), lambda: (0, 0))],
        out_specs=pl.BlockSpec((N, D), lambda: (0, 0)),
        out_shape=jax.ShapeDtypeStruct((N, D), jnp.float32),
    )(h)
    return pl.pallas_call(
        _fused_kernel,
        grid=(N // BLK,),
        in_specs=[
            pl.BlockSpec((N, D), lambda i: (0, 0)),
            pl.BlockSpec((BLK, D), lambda i: (i, 0)),
            pl.BlockSpec((N, D), lambda i: (0, 0)),
        ],
        out_specs=pl.BlockSpec((BLK, D), lambda i: (i, 0)),
        out_shape=jax.ShapeDtypeStruct((N, D), jnp.float32),
        compiler_params=pltpu.CompilerParams(
            dimension_semantics=("parallel",),
        ),
    )(hn, hn, h)


# speedup vs baseline: 1.1750x; 1.0482x over previous
import jax
import jax.numpy as jnp
from jax.experimental import pallas as pl
from jax.experimental.pallas import tpu as pltpu

N = 4096
D = 128
K = 10
THETA = 5.0
BLK = 1024
NEG = -3.0e38
LEAF_W = 512


def _distinct_maxima(s, k):
    out = []
    m = jnp.max(s, axis=1, keepdims=True)
    out.append(m)
    for _ in range(k - 1):
        m = jnp.max(jnp.where(s < m, s, NEG), axis=1, keepdims=True)
        out.append(m)
    return out


def _topk_candidates(s, k):
    if s.shape[1] > LEAF_W and k >= 2:
        half = s.shape[1] // 2
        a = s[:, :half]
        b = s[:, half:]
        hi = jnp.maximum(a, b)
        lo = jnp.minimum(a, b)
        return _topk_candidates(hi, k) + _topk_candidates(lo, (k + 1) // 2)
    return _distinct_maxima(s, k)


def _fused_kernel(h_ref, o_ref, hn_ref):
    i = pl.program_id(0)

    @pl.when(i == 0)
    def _():
        h0 = h_ref[...]
        norm = jnp.sqrt(jnp.sum(h0 * h0, axis=1, keepdims=True))
        hn_ref[...] = h0 / jnp.maximum(norm, 1e-8)

    hn = hn_ref[...]
    hn_blk = hn_ref[pl.ds(i * BLK, BLK), :]
    h = h_ref[...]

    s = jnp.dot(hn_blk, hn.T, preferred_element_type=jnp.float32)

    cand = jnp.concatenate(_topk_candidates(s, K), axis=1)
    t = _distinct_maxima(cand, K)[K - 1]

    e = jnp.where(s >= t, jnp.exp(THETA * s), 0.0)
    denom = jnp.sum(e, axis=1, keepdims=True)
    o_ref[...] = jnp.dot(e, h, preferred_element_type=jnp.float32) / denom


def kernel(h):
    return pl.pallas_call(
        _fused_kernel,
        grid=(N // BLK,),
        in_specs=[pl.BlockSpec((N, D), lambda i: (0, 0))],
        out_specs=pl.BlockSpec((BLK, D), lambda i: (i, 0)),
        out_shape=jax.ShapeDtypeStruct((N, D), jnp.float32),
        scratch_shapes=[pltpu.VMEM((N, D), jnp.float32)],
        compiler_params=pltpu.CompilerParams(
            dimension_semantics=("arbitrary",),
        ),
    )(h)


# speedup vs baseline: 1.1811x; 1.0051x over previous
import jax
import jax.numpy as jnp
from jax.experimental import pallas as pl
from jax.experimental.pallas import tpu as pltpu

N = 4096
D = 128
K = 10
THETA = 5.0
BLK = 1024
NEG = -3.0e38
LEAF_W = 512


def _distinct_maxima(s, k):
    out = []
    m = jnp.max(s, axis=1, keepdims=True)
    out.append(m)
    for _ in range(k - 1):
        m = jnp.max(jnp.where(s < m, s, NEG), axis=1, keepdims=True)
        out.append(m)
    return out


def _topk_candidates(s, k):
    if s.shape[1] > LEAF_W and k >= 5:
        half = s.shape[1] // 2
        a = s[:, :half]
        b = s[:, half:]
        hi = jnp.maximum(a, b)
        lo = jnp.minimum(a, b)
        return _topk_candidates(hi, k) + _topk_candidates(lo, (k + 1) // 2)
    return _distinct_maxima(s, k)


def _fused_kernel(h_ref, o_ref, hn_ref):
    i = pl.program_id(0)

    @pl.when(i == 0)
    def _():
        h0 = h_ref[...]
        norm = jnp.sqrt(jnp.sum(h0 * h0, axis=1, keepdims=True))
        hn_ref[...] = h0 / jnp.maximum(norm, 1e-8)

    hn = hn_ref[...]
    hn_blk = hn_ref[pl.ds(i * BLK, BLK), :]
    h = h_ref[...]

    s = jnp.dot(hn_blk, hn.T, preferred_element_type=jnp.float32)

    cand = jnp.concatenate(_topk_candidates(s, K), axis=1)
    t = _distinct_maxima(cand, K)[K - 1]

    e = jnp.where(s >= t, jnp.exp(THETA * s), 0.0)
    denom = jnp.sum(e, axis=1, keepdims=True)
    o_ref[...] = jnp.dot(e, h, preferred_element_type=jnp.float32) / denom


def kernel(h):
    return pl.pallas_call(
        _fused_kernel,
        grid=(N // BLK,),
        in_specs=[pl.BlockSpec((N, D), lambda i: (0, 0))],
        out_specs=pl.BlockSpec((BLK, D), lambda i: (i, 0)),
        out_shape=jax.ShapeDtypeStruct((N, D), jnp.float32),
        scratch_shapes=[pltpu.VMEM((N, D), jnp.float32)],
        compiler_params=pltpu.CompilerParams(
            dimension_semantics=("arbitrary",),
        ),
    )(h)
